# baseline (speedup 1.0000x reference)
import functools
import math

import jax
import jax.numpy as jnp
from jax import lax
from jax.experimental import pallas as pl
from jax.experimental.pallas import tpu as pltpu
from jax.experimental.pallas import tpu_sc as plsc

D_M = 64
SCALE = math.sqrt(D_M)
LANES = 16
CHUNK = 100
NBUF = 4
DCHUNK = 400


@functools.lru_cache(maxsize=None)
def _build_depad(vocab: int, num_cores: int, num_subcores: int):
    nw = num_cores * num_subcores
    n_ch = vocab // DCHUNK
    half = DCHUNK // 2
    mesh = plsc.VectorSubcoreMesh(core_axis_name="c", subcore_axis_name="s")
    assert vocab % DCHUNK == 0 and DCHUNK % 8 == 0

    @functools.partial(
        pl.kernel,
        mesh=mesh,
        out_type=jax.ShapeDtypeStruct((vocab // 2, 2 * D_M), jnp.float32),
        compiler_params=pltpu.CompilerParams(use_tc_tiling_on_sc=True),
        scratch_types=[
            pltpu.VMEM((DCHUNK, D_M), jnp.float32),
            pltpu.VMEM((2, half, 2 * D_M), jnp.float32),
            [pltpu.SemaphoreType.DMA] * 2,
        ],
    )
    def depad(tab_hbm, out_hbm, ibuf, obuf, ssems):
        wid = lax.axis_index("s") * num_cores + lax.axis_index("c")
        n_t = -(-n_ch // nw)

        @pl.loop(0, 2 * (-(-n_t // 2)), step=2)
        def outer(t0):
            for b in range(2):
                c = wid + nw * (t0 + b)

                @pl.when(c < n_ch)
                def _():
                    pltpu.sync_copy(
                        tab_hbm.at[pl.ds(c * DCHUNK, DCHUNK)], ibuf
                    )

                    @pl.when(t0 + b >= 2)
                    def _():
                        pltpu.make_async_copy(
                            obuf.at[b],
                            out_hbm.at[pl.ds(0, half)],
                            ssems[b],
                        ).wait()

                    def pair(q, c2):
                        for cc in range(D_M // LANES):
                            sl = pl.ds(cc * LANES, LANES)
                            obuf[b, q, sl] = ibuf[2 * q, sl]
                            sl2 = pl.ds(D_M + cc * LANES, LANES)
                            obuf[b, q, sl2] = ibuf[2 * q + 1, sl]
                        return c2

                    lax.fori_loop(0, half, pair, 0)
                    pltpu.async_copy(
                        obuf.at[b],
                        out_hbm.at[pl.ds(c * half, half)],
                        ssems[b],
                    )

        for b in range(2):
            pltpu.make_async_copy(
                obuf.at[b], out_hbm.at[pl.ds(0, half)], ssems[b]
            ).wait()

    return depad


@functools.lru_cache(maxsize=None)
def _build_emb(rows: int, cols: int, vocab: int, num_cores: int, num_subcores: int):
    nw = num_cores * num_subcores
    n_ch = rows * cols // (nw * CHUNK)
    mesh = plsc.VectorSubcoreMesh(core_axis_name="c", subcore_axis_name="s")
    assert n_ch % NBUF == 0 and n_ch >= 2 * NBUF

    @functools.partial(
        pl.kernel,
        mesh=mesh,
        out_type=jax.ShapeDtypeStruct((rows * cols, 2 * D_M), jnp.float32),
        compiler_params=pltpu.CompilerParams(use_tc_tiling_on_sc=False),
        scratch_types=[
            pltpu.VMEM((n_ch, CHUNK), jnp.int32),
            pltpu.VMEM((NBUF, CHUNK, D_M), jnp.float32),
            pltpu.VMEM((NBUF, CHUNK, D_M), jnp.float32),
            [pltpu.SemaphoreType.DMA] * NBUF,
            [pltpu.SemaphoreType.DMA] * NBUF,
        ],
    )
    def emb(x_hbm, tab_hbm, out_hbm, idx_v, gbuf, sbuf, gsems, ssems):
        wid = lax.axis_index("s") * num_cores + lax.axis_index("c")
        base = wid * n_ch
        pltpu.sync_copy(x_hbm.at[pl.ds(base, n_ch)], idx_v)

        for b in range(NBUF):
            pltpu.async_copy(tab_hbm.at[idx_v.at[b]], gbuf.at[b], gsems[b])

        def scale(b):
            def row(r, c2):
                for c in range(D_M // LANES):
                    sl = pl.ds(c * LANES, LANES)
                    sbuf[b, r, sl] = gbuf[b, r, sl] * SCALE
                return c2

            lax.fori_loop(0, CHUNK, row, 0)

        @pl.loop(0, n_ch, step=NBUF)
        def outer(j0):
            for b in range(NBUF):
                k = j0 + b
                pltpu.make_async_copy(
                    tab_hbm.at[pl.ds(0, CHUNK)], gbuf.at[b], gsems[b]
                ).wait()
                @pl.when(j0 > 0)
                def _():
                    pltpu.make_async_copy(
                        sbuf.at[b],
                        out_hbm.at[pl.ds(0, CHUNK), pl.ds(0, D_M)],
                        ssems[b],
                    ).wait()

                scale(b)
                @pl.when(k + NBUF < n_ch)
                def _():
                    pltpu.async_copy(
                        tab_hbm.at[idx_v.at[k + NBUF]], gbuf.at[b], gsems[b]
                    )

                pltpu.async_copy(
                    sbuf.at[b],
                    out_hbm.at[pl.ds((base + k) * CHUNK, CHUNK), pl.ds(0, D_M)],
                    ssems[b],
                )

        for b in range(NBUF):
            pltpu.make_async_copy(
                sbuf.at[b],
                out_hbm.at[pl.ds(0, CHUNK), pl.ds(0, D_M)],
                ssems[b],
            ).wait()

    return emb


def kernel(x, table):
    rows, cols = x.shape
    vocab = table.shape[0]
    info = plsc.get_sparse_core_info()
    nw = info.num_cores * info.num_subcores
    assert (rows * cols) % (nw * CHUNK) == 0
    n_ch = rows * cols // (nw * CHUNK)
    xf = x.reshape(nw * n_ch, CHUNK).astype(jnp.int32)
    depad = _build_depad(vocab, info.num_cores, info.num_subcores)
    tab_flat = depad(table).reshape(vocab, D_M)
    emb = _build_emb(rows, cols, vocab, info.num_cores, info.num_subcores)
    out2 = emb(xf, tab_flat)
    return out2[:, :D_M].reshape(rows, cols, D_M)


# speedup vs baseline: 1.4740x; 1.4740x over previous
import functools
import math

import jax
import jax.numpy as jnp
from jax import lax
from jax.experimental import pallas as pl
from jax.experimental.pallas import tpu as pltpu
from jax.experimental.pallas import tpu_sc as plsc

D_M = 64
SCALE = math.sqrt(D_M)
LANES = 16
CHUNK = 100
NBUF = 4


@functools.lru_cache(maxsize=None)
def _build_emb(rows: int, cols: int, vocab: int, num_cores: int, num_subcores: int):
    nw = num_cores * num_subcores
    n_ch = rows * cols // (nw * CHUNK)
    mesh = plsc.VectorSubcoreMesh(core_axis_name="c", subcore_axis_name="s")
    assert n_ch % NBUF == 0 and n_ch >= 2 * NBUF

    @functools.partial(
        pl.kernel,
        mesh=mesh,
        out_type=jax.ShapeDtypeStruct((rows * cols, 2 * D_M), jnp.float32),
        compiler_params=pltpu.CompilerParams(use_tc_tiling_on_sc=False),
        scratch_types=[
            pltpu.VMEM((n_ch, CHUNK), jnp.int32),
            pltpu.VMEM((NBUF, CHUNK, D_M), jnp.float32),
            pltpu.VMEM((NBUF, CHUNK, D_M), jnp.float32),
            [pltpu.SemaphoreType.DMA] * NBUF,
            [pltpu.SemaphoreType.DMA] * NBUF,
        ],
    )
    def emb(x_hbm, tab_hbm, out_hbm, idx_v, gbuf, sbuf, gsems, ssems):
        wid = lax.axis_index("s") * num_cores + lax.axis_index("c")
        base = wid * n_ch
        pltpu.sync_copy(x_hbm.at[pl.ds(base, n_ch)], idx_v)

        for b in range(NBUF):
            pltpu.async_copy(tab_hbm.at[idx_v.at[b]], gbuf.at[b], gsems[b])

        def scale(b):
            def row(r, c2):
                for c in range(D_M // LANES):
                    sl = pl.ds(c * LANES, LANES)
                    sbuf[b, r, sl] = gbuf[b, r, sl] * SCALE
                return c2

            lax.fori_loop(0, CHUNK, row, 0)

        @pl.loop(0, n_ch, step=NBUF)
        def outer(j0):
            for b in range(NBUF):
                k = j0 + b
                pltpu.make_async_copy(
                    tab_hbm.at[pl.ds(0, CHUNK)], gbuf.at[b], gsems[b]
                ).wait()
                @pl.when(j0 > 0)
                def _():
                    pltpu.make_async_copy(
                        sbuf.at[b],
                        out_hbm.at[pl.ds(0, CHUNK), pl.ds(0, D_M)],
                        ssems[b],
                    ).wait()

                scale(b)
                @pl.when(k + NBUF < n_ch)
                def _():
                    pltpu.async_copy(
                        tab_hbm.at[idx_v.at[k + NBUF]], gbuf.at[b], gsems[b]
                    )

                pltpu.async_copy(
                    sbuf.at[b],
                    out_hbm.at[pl.ds((base + k) * CHUNK, CHUNK), pl.ds(0, D_M)],
                    ssems[b],
                )

        for b in range(NBUF):
            pltpu.make_async_copy(
                sbuf.at[b],
                out_hbm.at[pl.ds(0, CHUNK), pl.ds(0, D_M)],
                ssems[b],
            ).wait()

    return emb


def kernel(x, table):
    rows, cols = x.shape
    vocab = table.shape[0]
    info = plsc.get_sparse_core_info()
    nw = info.num_cores * info.num_subcores
    assert (rows * cols) % (nw * CHUNK) == 0
    n_ch = rows * cols // (nw * CHUNK)
    xf = x.reshape(nw * n_ch, CHUNK).astype(jnp.int32)
    emb = _build_emb(rows, cols, vocab, info.num_cores, info.num_subcores)
    out2 = emb(xf, table)
    return out2[:, :D_M].reshape(rows, cols, D_M)
